# Initial kernel scaffold; baseline (speedup 1.0000x reference)
import jax
import jax.numpy as jnp
from jax.experimental import pallas as pl


def kernel(x, index):
    raise NotImplementedError("write your pallas kernel here")


# speedup vs baseline: 8.4744x; 8.4744x over previous
import functools

import jax
import jax.numpy as jnp
from jax import lax
from jax.experimental import pallas as pl
from jax.experimental.pallas import tpu as pltpu
from jax.experimental.pallas import tpu_sc as plsc

E = 320000
D = 128
N = 10000

NC = 2
NS = 16
NW = NC * NS
RPW = E // NW
CHUNK = 40
NCHUNK = RPW // CHUNK
NBUF = 5
NP = 10240
NPT = NP // NS
ZROWS = 16


def _sc_partial_sums(x, index):
    mesh = plsc.VectorSubcoreMesh(core_axis_name="c", subcore_axis_name="s")
    scratch = (
        [pltpu.VMEM((CHUNK, D), jnp.float32) for _ in range(NBUF)]
        + [pltpu.VMEM((CHUNK,), jnp.int32) for _ in range(NBUF)]
        + [pltpu.VMEM((ZROWS, D), jnp.float32)]
        + [pltpu.VMEM_SHARED((NP, D), jnp.float32)]
        + [pltpu.SemaphoreType.DMA for _ in range(2 * NBUF)]
    )

    @functools.partial(
        pl.kernel,
        out_type=jax.ShapeDtypeStruct((NC * NP, D), jnp.float32),
        mesh=mesh,
        scratch_types=scratch,
    )
    def k(x_hbm, idx_hbm, out_hbm, *refs):
        xbufs = refs[0:NBUF]
        ibufs = refs[NBUF:2 * NBUF]
        zbuf = refs[2 * NBUF]
        acc = refs[2 * NBUF + 1]
        xsems = refs[2 * NBUF + 2:2 * NBUF + 2 + NBUF]
        isems = refs[2 * NBUF + 2 + NBUF:2 * NBUF + 2 + 2 * NBUF]

        cid = lax.axis_index("c")
        sid = lax.axis_index("s")
        row0 = (cid * NS + sid) * RPW

        def start_load(c, b):
            base = row0 + c * CHUNK
            pltpu.async_copy(x_hbm.at[pl.ds(base, CHUNK)], xbufs[b], xsems[b])
            pltpu.async_copy(idx_hbm.at[pl.ds(base, CHUNK)], ibufs[b], isems[b])

        def wait_load(b):
            pltpu.make_async_copy(x_hbm.at[pl.ds(0, CHUNK)], xbufs[b], xsems[b]).wait()
            pltpu.make_async_copy(idx_hbm.at[pl.ds(0, CHUNK)], ibufs[b], isems[b]).wait()

        for b in range(NBUF):
            start_load(b, b)

        zero = jnp.zeros((16,), jnp.float32)

        def zrow(i, carry):
            for j in range(D // 16):
                zbuf[i, pl.ds(j * 16, 16)] = zero
            return carry

        lax.fori_loop(0, ZROWS, zrow, 0)
        for t in range(NPT // ZROWS):
            pltpu.sync_copy(zbuf, acc.at[pl.ds(sid * NPT + t * ZROWS, ZROWS)])
        plsc.subcore_barrier()

        def group(g, carry):
            c0 = g * NBUF
            for b in range(NBUF):
                c = c0 + b
                wait_load(b)
                pltpu.sync_copy(xbufs[b], acc.at[ibufs[b]], add=True)

                @pl.when(c + NBUF < NCHUNK)
                def _():
                    start_load(c + NBUF, b)

            return carry

        lax.fori_loop(0, NCHUNK // NBUF, group, 0)

        plsc.subcore_barrier()
        pltpu.sync_copy(
            acc.at[pl.ds(sid * NPT, NPT)],
            out_hbm.at[pl.ds(cid * NP + sid * NPT, NPT)],
        )

    return k(x, index).reshape(NC, NP, D)


def _combine(p):
    blk = N // 5

    def add_body(a_ref, b_ref, o_ref):
        o_ref[...] = a_ref[0] + b_ref[0]

    return pl.pallas_call(
        add_body,
        grid=(5,),
        in_specs=[
            pl.BlockSpec((1, blk, D), lambda i: (0, i, 0)),
            pl.BlockSpec((1, blk, D), lambda i: (1, i, 0)),
        ],
        out_specs=pl.BlockSpec((blk, D), lambda i: (i, 0)),
        out_shape=jax.ShapeDtypeStruct((N, D), jnp.float32),
    )(p, p)


def kernel(x, index):
    p = _sc_partial_sums(x, index)
    return _combine(p)
